# Initial kernel scaffold; baseline (speedup 1.0000x reference)
import jax
import jax.numpy as jnp
from jax.experimental import pallas as pl


def kernel(atom_features, edge_index, edge_attr, W1, b1, W2, b2, Watt, batt, Wa, ba, Wih, Whh, bih, bhh):
    raise NotImplementedError("write your pallas kernel here")


# speedup vs baseline: 5.0534x; 5.0534x over previous
import functools

import jax
import jax.numpy as jnp
from jax import lax
from jax.experimental import pallas as pl
from jax.experimental.pallas import tpu as pltpu
from jax.experimental.pallas import tpu_sc as plsc

N = 10000
E = 320000
D = 128
DE = 16
H = 128
LEAKY = 0.2
ROW = H + 16
NC = 2
NS = 16
NW = NC * NS
EPW = E // NW
C = 80
NCHUNK = EPW // C
NGRP = C // 16
NP = 10240
TPR = NP // NS

_HI = jax.lax.Precision.HIGHEST


def _leaky(v):
    return jnp.maximum(v, LEAKY * v)


def _dot_t(a, b):
    return lax.dot_general(a, b, (((1,), (1,)), ((), ())),
                           precision=_HI, preferred_element_type=jnp.float32)


def _node_body(x_ref, w1_ref, b1_ref, wa_ref, ba_ref, watt_ref, batt_ref,
               w2_ref, h_ref, p_ref, t_ref, s1r_ref):
    x = x_ref[...]
    h = _leaky(_dot_t(x, w1_ref[...]) + b1_ref[...])
    h_ref[...] = h
    t_ref[...] = _dot_t(h, wa_ref[...]) + ba_ref[...]
    p_ref[...] = _dot_t(x, w2_ref[:, :D])
    s1r_ref[...] = _dot_t(h, watt_ref[...]) + batt_ref[0, 0]


def _node_kernel(x, W1, b1r, Wa, bar, Watt, battr, W2):
    R = 1000
    g = N // R
    return pl.pallas_call(
        _node_body,
        grid=(g,),
        in_specs=[
            pl.BlockSpec((R, D), lambda i: (i, 0)),
            pl.BlockSpec((H, D), lambda i: (0, 0)),
            pl.BlockSpec((1, H), lambda i: (0, 0)),
            pl.BlockSpec((H, H), lambda i: (0, 0)),
            pl.BlockSpec((1, H), lambda i: (0, 0)),
            pl.BlockSpec((16, H), lambda i: (0, 0)),
            pl.BlockSpec((1, 1), lambda i: (0, 0)),
            pl.BlockSpec((H, D + DE), lambda i: (0, 0)),
        ],
        out_specs=[
            pl.BlockSpec((R, H), lambda i: (i, 0)),
            pl.BlockSpec((R, H), lambda i: (i, 0)),
            pl.BlockSpec((R, H), lambda i: (i, 0)),
            pl.BlockSpec((R, 16), lambda i: (i, 0)),
        ],
        out_shape=[
            jax.ShapeDtypeStruct((N, H), jnp.float32),
            jax.ShapeDtypeStruct((N, H), jnp.float32),
            jax.ShapeDtypeStruct((N, H), jnp.float32),
            jax.ShapeDtypeStruct((N, 16), jnp.float32),
        ],
    )(x, W1, b1r, Wa, bar, Watt, battr, W2)


def _q_body(ea_ref, w2_ref, b2_ref, q_ref):
    q_ref[...] = _dot_t(ea_ref[...], w2_ref[:, D:]) + b2_ref[...]


def _q_kernel(ea, W2, b2r):
    R = 2000
    g = E // R
    return pl.pallas_call(
        _q_body,
        grid=(g,),
        in_specs=[
            pl.BlockSpec((R, DE), lambda i: (i, 0)),
            pl.BlockSpec((H, D + DE), lambda i: (0, 0)),
            pl.BlockSpec((1, H), lambda i: (0, 0)),
        ],
        out_specs=pl.BlockSpec((R, H), lambda i: (i, 0)),
        out_shape=jax.ShapeDtypeStruct((E, H), jnp.float32),
    )(ea, W2, b2r)


def _sc_edge_body(p_hbm, t_hbm, s1r_hbm, q_hbm, ei_hbm, wb_hbm, out_hbm,
                  src_v, dst_v, rows_v, q_v, s1_v, w_v, out_v, wb_v, acc_sh):
    cid = lax.axis_index("c")
    sid = lax.axis_index("s")
    wid = sid * NC + cid
    lane = lax.iota(jnp.int32, 16)

    def _zrow(i, _):
        for k in range(ROW // 16):
            out_v[i, pl.ds(k * 16, 16)] = jnp.zeros((16,), jnp.float32)
        return _
    lax.fori_loop(0, C, _zrow, None)
    for k in range(TPR // C):
        pltpu.sync_copy(out_v, acc_sh.at[pl.ds(sid * TPR + k * C, C)])
    pltpu.sync_copy(wb_hbm, wb_v)
    plsc.subcore_barrier()

    base = wid * EPW

    def _chunk(ci, _):
        e0 = base + ci * C
        pltpu.sync_copy(ei_hbm.at[0, pl.ds(e0, C)], src_v)
        pltpu.sync_copy(ei_hbm.at[1, pl.ds(e0, C)], dst_v)
        pltpu.sync_copy(p_hbm.at[src_v], rows_v)
        pltpu.sync_copy(s1r_hbm.at[dst_v], s1_v)
        pltpu.sync_copy(q_hbm.at[pl.ds(e0, C)], q_v)

        def _score(g, _g):
            gbase = g * 16
            s2v = jnp.zeros((16,), jnp.float32)
            for j in range(16):
                e = gbase + j
                acc = jnp.zeros((16,), jnp.float32)
                for r in range(H // 16):
                    u = rows_v[e, pl.ds(r * 16, 16)] + q_v[e, pl.ds(r * 16, 16)]
                    acc = acc + _leaky(u) * wb_v[pl.ds(r * 16, 16)]
                s2v = jnp.where(lane == j, jnp.sum(acc), s2v)
            s1g = plsc.load_gather(s1_v, [lane + gbase, jnp.zeros((16,), jnp.int32)])
            pre = s1g + s2v
            w_v[pl.ds(gbase, 16)] = jnp.exp(_leaky(pre))
            return _g
        lax.fori_loop(0, NGRP, _score, None)

        pltpu.sync_copy(t_hbm.at[src_v], rows_v)

        def _message(g, _g):
            gbase = g * 16
            w = w_v[pl.ds(gbase, 16)]
            for j in range(16):
                e = gbase + j
                wj = jnp.sum(jnp.where(lane == j, w, 0.0))
                for r in range(H // 16):
                    out_v[e, pl.ds(r * 16, 16)] = rows_v[e, pl.ds(r * 16, 16)] * wj
                out_v[e, pl.ds(H, 16)] = jnp.zeros((16,), jnp.float32)
            plsc.store_scatter(out_v, [lane + gbase, jnp.full((16,), H, jnp.int32)], w)
            return _g
        lax.fori_loop(0, NGRP, _message, None)
        pltpu.sync_copy(out_v, acc_sh.at[dst_v], add=True)
        return _
    lax.fori_loop(0, NCHUNK, _chunk, None)

    plsc.subcore_barrier()
    for k in range(TPR // C):
        r0 = sid * TPR + k * C
        pltpu.sync_copy(acc_sh.at[pl.ds(r0, C)], out_v)
        pltpu.sync_copy(out_v, out_hbm.at[cid, pl.ds(r0, C)])


def _sc_edge_kernel(p, t, s1r, q, ei, wb):
    mesh = plsc.VectorSubcoreMesh(core_axis_name="c", subcore_axis_name="s")
    f = functools.partial(
        pl.kernel, mesh=mesh,
        compiler_params=pltpu.CompilerParams(use_tc_tiling_on_sc=False,
                                             needs_layout_passes=False),
        out_type=jax.ShapeDtypeStruct((NC, NP, ROW), jnp.float32),
        scratch_types=[
            pltpu.VMEM((C,), jnp.int32),
            pltpu.VMEM((C,), jnp.int32),
            pltpu.VMEM((C, H), jnp.float32),
            pltpu.VMEM((C, H), jnp.float32),
            pltpu.VMEM((C, 16), jnp.float32),
            pltpu.VMEM((C,), jnp.float32),
            pltpu.VMEM((C, ROW), jnp.float32),
            pltpu.VMEM((H,), jnp.float32),
            pltpu.VMEM_SHARED((NP, ROW), jnp.float32),
        ],
    )(_sc_edge_body)
    return f(p, t, s1r, q, ei, wb)


def _post_body(acc_ref, h_ref, wih_ref, whh_ref, bih_ref, bhh_ref, out_ref):
    a = acc_ref[...]
    s = a[0] + a[1]
    numer = s[:, :H]
    denom = s[:, H:H + 1]
    transform = numer / jnp.maximum(denom, 1e-16)
    context = jnp.where(transform > 0, transform,
                        jnp.exp(jnp.minimum(transform, 0.0)) - 1.0)
    h = h_ref[...]
    oe = jnp.where(denom > 0, context, h)
    gi = _dot_t(oe, wih_ref[...]) + bih_ref[...]
    gh = _dot_t(h, whh_ref[...]) + bhh_ref[...]
    r = jax.nn.sigmoid(gi[:, :H] + gh[:, :H])
    z = jax.nn.sigmoid(gi[:, H:2 * H] + gh[:, H:2 * H])
    n = jnp.tanh(gi[:, 2 * H:] + r * gh[:, 2 * H:])
    out_ref[...] = (1.0 - z) * n + z * h


def _post_kernel(acc, h, Wih, Whh, bihr, bhhr):
    R = 1000
    g = N // R
    return pl.pallas_call(
        _post_body,
        grid=(g,),
        in_specs=[
            pl.BlockSpec((NC, R, ROW), lambda i: (0, i, 0)),
            pl.BlockSpec((R, H), lambda i: (i, 0)),
            pl.BlockSpec((3 * H, H), lambda i: (0, 0)),
            pl.BlockSpec((3 * H, H), lambda i: (0, 0)),
            pl.BlockSpec((1, 3 * H), lambda i: (0, 0)),
            pl.BlockSpec((1, 3 * H), lambda i: (0, 0)),
        ],
        out_specs=pl.BlockSpec((R, H), lambda i: (i, 0)),
        out_shape=jax.ShapeDtypeStruct((N, H), jnp.float32),
    )(acc, h, Wih, Whh, bihr, bhhr)


def kernel(atom_features, edge_index, edge_attr, W1, b1, W2, b2, Watt, batt,
           Wa, ba, Wih, Whh, bih, bhh):
    b1r = b1.reshape(1, H)
    bar = ba.reshape(1, H)
    b2r = b2.reshape(1, H)
    battr = batt.reshape(1, 1)
    bihr = bih.reshape(1, 3 * H)
    bhhr = bhh.reshape(1, 3 * H)
    wb = Watt[0, H:]
    watt16 = jnp.broadcast_to(Watt[:, :H], (16, H))

    h, p, t, s1r = _node_kernel(atom_features, W1, b1r, Wa, bar, watt16, battr, W2)
    q = _q_kernel(edge_attr, W2, b2r)
    acc = _sc_edge_kernel(p, t, s1r, q, edge_index, wb)
    new_h = _post_kernel(acc, h, Wih, Whh, bihr, bhhr)
    return (new_h, h)
